# Initial kernel scaffold; baseline (speedup 1.0000x reference)
import jax
import jax.numpy as jnp
from jax.experimental import pallas as pl


def kernel(coords, features, W, bias):
    raise NotImplementedError("write your pallas kernel here")


# speedup vs baseline: 1.8819x; 1.8819x over previous
import jax
import jax.numpy as jnp
from jax.experimental import pallas as pl

_K = 16


def _locse_kernel(q_ref, ct_ref, f_ref, w_ref, b_ref, o_ref):
    q = q_ref[0]
    ct = ct_ref[0]
    Q = q.shape[0]
    N = ct.shape[1]

    qsq = jnp.sum(q * q, axis=1, keepdims=True)
    csq = jnp.sum(ct * ct, axis=0, keepdims=True)
    cx = ct[0:1, :]
    cy = ct[1:2, :]
    cz = ct[2:3, :]
    qc = jax.lax.dot_general(
        q, ct, (((1,), (0,)), ((), ())),
        preferred_element_type=jnp.float32)
    d2 = qsq - 2.0 * qc + csq

    iota = jax.lax.broadcasted_iota(jnp.int32, (Q, N), 1)

    nxs, nys, nzs = [], [], []
    for _ in range(_K):
        m = jnp.min(d2, axis=1, keepdims=True)
        ism = d2 <= m
        idx = jnp.min(jnp.where(ism, iota, N), axis=1, keepdims=True)
        onehot = iota == idx
        nxs.append(jnp.sum(jnp.where(onehot, cx, 0.0), axis=1, keepdims=True))
        nys.append(jnp.sum(jnp.where(onehot, cy, 0.0), axis=1, keepdims=True))
        nzs.append(jnp.sum(jnp.where(onehot, cz, 0.0), axis=1, keepdims=True))
        d2 = jnp.where(onehot, jnp.float32(jnp.inf), d2)

    nbx = jnp.concatenate(nxs, axis=1)
    nby = jnp.concatenate(nys, axis=1)
    nbz = jnp.concatenate(nzs, axis=1)

    ex = jnp.broadcast_to(q[:, 0:1], (Q, _K))
    ey = jnp.broadcast_to(q[:, 1:2], (Q, _K))
    ez = jnp.broadcast_to(q[:, 2:3], (Q, _K))
    dx, dy, dz = ex - nbx, ey - nby, ez - nbz
    dist = jnp.sqrt(jnp.maximum(dx * dx + dy * dy + dz * dz, 1e-12))

    spatial = jnp.stack(
        [ex, ey, ez, nbx, nby, nbz, dx, dy, dz, dist],
        axis=0)
    spatial = spatial.reshape(10, Q * _K)

    w = w_ref[...]
    mlp = jax.lax.dot_general(
        w, spatial, (((1,), (0,)), ((), ())),
        preferred_element_type=jnp.float32) + b_ref[...]

    f = f_ref[0]
    featb = jnp.broadcast_to(f[:, :, None], f.shape + (_K,)).reshape(
        f.shape[0], Q * _K)

    o_ref[0] = jnp.concatenate([mlp, featb], axis=0)


def kernel(coords, features, W, bias):
    B, N, _ = coords.shape
    d = features.shape[1]
    d_out = W.shape[0]
    Q = 128
    ct = jnp.transpose(coords, (0, 2, 1))
    f2 = features[:, :, :, 0]
    b2 = bias[:, None]
    out = pl.pallas_call(
        _locse_kernel,
        grid=(B, N // Q),
        in_specs=[
            pl.BlockSpec((1, Q, 3), lambda b, i: (b, i, 0)),
            pl.BlockSpec((1, 3, N), lambda b, i: (b, 0, 0)),
            pl.BlockSpec((1, d, Q), lambda b, i: (b, 0, i)),
            pl.BlockSpec((d_out, 10), lambda b, i: (0, 0)),
            pl.BlockSpec((d_out, 1), lambda b, i: (0, 0)),
        ],
        out_specs=pl.BlockSpec((1, d_out + d, Q * _K), lambda b, i: (b, 0, i)),
        out_shape=jax.ShapeDtypeStruct((B, d_out + d, N * _K), jnp.float32),
    )(coords, ct, f2, W, b2)
    return out.reshape(B, d_out + d, N, _K)


# speedup vs baseline: 2.9278x; 1.5558x over previous
import jax
import jax.numpy as jnp
from jax.experimental import pallas as pl

_K = 16


def _locse_kernel(q_ref, ct_ref, f_ref, w_ref, b_ref, o_ref):
    q = q_ref[0]
    ct = ct_ref[0]
    Q = q.shape[0]
    N = ct.shape[1]

    qsq = jnp.sum(q * q, axis=1, keepdims=True)
    csq = jnp.sum(ct * ct, axis=0, keepdims=True)
    cx = ct[0:1, :]
    cy = ct[1:2, :]
    cz = ct[2:3, :]
    qc = jax.lax.dot_general(
        q, ct, (((1,), (0,)), ((), ())),
        preferred_element_type=jnp.float32)
    d2 = qsq - 2.0 * qc + csq

    inf = jnp.float32(jnp.inf)
    nxs, nys, nzs = [], [], []
    for _ in range(_K):
        m = jnp.min(d2, axis=1, keepdims=True)
        oh = d2 <= m
        nxs.append(jnp.min(jnp.where(oh, cx, inf), axis=1, keepdims=True))
        nys.append(jnp.min(jnp.where(oh, cy, inf), axis=1, keepdims=True))
        nzs.append(jnp.min(jnp.where(oh, cz, inf), axis=1, keepdims=True))
        d2 = jnp.where(oh, inf, d2)

    nbx = jnp.concatenate(nxs, axis=1)
    nby = jnp.concatenate(nys, axis=1)
    nbz = jnp.concatenate(nzs, axis=1)

    ex = jnp.broadcast_to(q[:, 0:1], (Q, _K))
    ey = jnp.broadcast_to(q[:, 1:2], (Q, _K))
    ez = jnp.broadcast_to(q[:, 2:3], (Q, _K))
    dx, dy, dz = ex - nbx, ey - nby, ez - nbz
    dist = jnp.sqrt(jnp.maximum(dx * dx + dy * dy + dz * dz, 1e-12))

    spatial = jnp.stack(
        [ex, ey, ez, nbx, nby, nbz, dx, dy, dz, dist],
        axis=0)
    spatial = spatial.reshape(10, Q * _K)

    w = w_ref[...]
    mlp = jax.lax.dot_general(
        w, spatial, (((1,), (0,)), ((), ())),
        preferred_element_type=jnp.float32) + b_ref[...]

    f = f_ref[0]
    featb = jnp.broadcast_to(f[:, :, None], f.shape + (_K,)).reshape(
        f.shape[0], Q * _K)

    o_ref[0] = jnp.concatenate([mlp, featb], axis=0)


def kernel(coords, features, W, bias):
    B, N, _ = coords.shape
    d = features.shape[1]
    d_out = W.shape[0]
    Q = 128
    ct = jnp.transpose(coords, (0, 2, 1))
    f2 = features[:, :, :, 0]
    b2 = bias[:, None]
    out = pl.pallas_call(
        _locse_kernel,
        grid=(B, N // Q),
        in_specs=[
            pl.BlockSpec((1, Q, 3), lambda b, i: (b, i, 0)),
            pl.BlockSpec((1, 3, N), lambda b, i: (b, 0, 0)),
            pl.BlockSpec((1, d, Q), lambda b, i: (b, 0, i)),
            pl.BlockSpec((d_out, 10), lambda b, i: (0, 0)),
            pl.BlockSpec((d_out, 1), lambda b, i: (0, 0)),
        ],
        out_specs=pl.BlockSpec((1, d_out + d, Q * _K), lambda b, i: (b, 0, i)),
        out_shape=jax.ShapeDtypeStruct((B, d_out + d, N * _K), jnp.float32),
    )(coords, ct, f2, W, b2)
    return out.reshape(B, d_out + d, N, _K)


# speedup vs baseline: 3.0503x; 1.0418x over previous
import jax
import jax.numpy as jnp
from jax.experimental import pallas as pl

_K = 16


def _locse_kernel(q_ref, ct_ref, f_ref, w_ref, b_ref, o_ref):
    q = q_ref[0]
    ct = ct_ref[0]
    Q = q.shape[0]
    N = ct.shape[1]

    qsq = jnp.sum(q * q, axis=1, keepdims=True)
    csq = jnp.sum(ct * ct, axis=0, keepdims=True)
    cx = ct[0:1, :]
    cy = ct[1:2, :]
    cz = ct[2:3, :]
    qc = jax.lax.dot_general(
        q, ct, (((1,), (0,)), ((), ())),
        preferred_element_type=jnp.float32)
    d2 = qsq - 2.0 * qc + csq

    inf = jnp.float32(jnp.inf)
    nxs, nys, nzs = [], [], []
    for _ in range(_K):
        m = jnp.min(d2, axis=1, keepdims=True)
        oh = d2 <= m
        nxs.append(jnp.min(jnp.where(oh, cx, inf), axis=1, keepdims=True))
        nys.append(jnp.min(jnp.where(oh, cy, inf), axis=1, keepdims=True))
        nzs.append(jnp.min(jnp.where(oh, cz, inf), axis=1, keepdims=True))
        d2 = jnp.where(oh, inf, d2)

    nbx = jnp.concatenate(nxs, axis=1)
    nby = jnp.concatenate(nys, axis=1)
    nbz = jnp.concatenate(nzs, axis=1)

    ex = jnp.broadcast_to(q[:, 0:1], (Q, _K))
    ey = jnp.broadcast_to(q[:, 1:2], (Q, _K))
    ez = jnp.broadcast_to(q[:, 2:3], (Q, _K))
    dx, dy, dz = ex - nbx, ey - nby, ez - nbz
    dist = jnp.sqrt(jnp.maximum(dx * dx + dy * dy + dz * dz, 1e-12))

    spatial = jnp.stack(
        [ex, ey, ez, nbx, nby, nbz, dx, dy, dz, dist],
        axis=0)
    spatial = spatial.reshape(10, Q * _K)

    w = w_ref[...]
    mlp = jax.lax.dot_general(
        w, spatial, (((1,), (0,)), ((), ())),
        preferred_element_type=jnp.float32) + b_ref[...]

    f = f_ref[0]
    featb = jnp.broadcast_to(f[:, :, None], f.shape + (_K,)).reshape(
        f.shape[0], Q * _K)

    o_ref[0] = jnp.concatenate([mlp, featb], axis=0)


def kernel(coords, features, W, bias):
    B, N, _ = coords.shape
    d = features.shape[1]
    d_out = W.shape[0]
    Q = 256
    ct = jnp.transpose(coords, (0, 2, 1))
    f2 = features[:, :, :, 0]
    b2 = bias[:, None]
    out = pl.pallas_call(
        _locse_kernel,
        grid=(B, N // Q),
        in_specs=[
            pl.BlockSpec((1, Q, 3), lambda b, i: (b, i, 0)),
            pl.BlockSpec((1, 3, N), lambda b, i: (b, 0, 0)),
            pl.BlockSpec((1, d, Q), lambda b, i: (b, 0, i)),
            pl.BlockSpec((d_out, 10), lambda b, i: (0, 0)),
            pl.BlockSpec((d_out, 1), lambda b, i: (0, 0)),
        ],
        out_specs=pl.BlockSpec((1, d_out + d, Q * _K), lambda b, i: (b, 0, i)),
        out_shape=jax.ShapeDtypeStruct((B, d_out + d, N * _K), jnp.float32),
    )(coords, ct, f2, W, b2)
    return out.reshape(B, d_out + d, N, _K)
